# baseline (speedup 1.0000x reference)
import functools

import jax
import jax.numpy as jnp
from jax.experimental import pallas as pl
from jax.experimental.pallas import tpu as pltpu

_CS = 512
_TN = 2000
_G = 64
_LANES = 128


def _transform_krn(x_ref, wr_ref, o_ref):
    o_ref[0] = jnp.dot(x_ref[...], wr_ref[0], preferred_element_type=jnp.float32)


def _root_krn(x_ref, w_ref, b_ref, o_ref):
    o_ref[...] = (
        jnp.dot(x_ref[...], w_ref[...], preferred_element_type=jnp.float32)
        + b_ref[...]
    )


def _count_krn(nch, dst_ref, et_ref, w0_ref, cnt_ref):
    c = pl.program_id(0)

    @pl.when(c == 0)
    def _():
        cnt_ref[...] = jnp.zeros_like(cnt_ref)

    lanes = jax.lax.broadcasted_iota(jnp.int32, (1, _LANES), 1)

    def body(i, _):
        d = dst_ref[0, 0, i]
        t = et_ref[0, 0, i]
        w0 = w0_ref[0, 0, i]
        row = cnt_ref[pl.ds(d, 1), :]
        cnt_ref[pl.ds(d, 1), :] = row + jnp.where(lanes == t, w0, 0.0)
        return 0

    jax.lax.fori_loop(0, _CS, body, 0)


def _scatter_krn(nch, rel_ref, hall_ref, out0_ref, cnt_ref, src_ref, dst_ref,
                 et_ref, w0_ref, out_ref):
    c = pl.program_id(0)

    @pl.when(c == 0)
    def _():
        out_ref[...] = out0_ref[...]

    lanes = jax.lax.broadcasted_iota(jnp.int32, (1, _LANES), 1)

    def body(i, _):
        s = src_ref[0, 0, i]
        d = dst_ref[0, 0, i]
        t = et_ref[0, 0, i]
        w0 = w0_ref[0, 0, i]
        crow = cnt_ref[pl.ds(d, 1), :]
        cv = jnp.sum(jnp.where(lanes == t, crow, 0.0))
        w = w0 / jnp.maximum(cv, 1.0)
        out_ref[pl.ds(d, 1), :] = (
            out_ref[pl.ds(d, 1), :] + hall_ref[0, pl.ds(s, 1), :] * w
        )
        return 0

    jax.lax.fori_loop(0, _CS, body, 0)

    @pl.when(c == nch - 1)
    def _():
        out_ref[...] = jnp.maximum(out_ref[...], 0.0)


def _pool_krn(h_ref, b_ref, p_ref):
    g = pl.program_id(0)
    mask = b_ref[:, 0:1] == g
    p_ref[pl.ds(g, 1), :] = jnp.max(
        jnp.where(mask, h_ref[...], -jnp.inf), axis=0, keepdims=True
    )


def _transform(x, Wr):
    R, F, H = Wr.shape
    N = x.shape[0]
    return pl.pallas_call(
        _transform_krn,
        grid=(R, N // _TN),
        in_specs=[
            pl.BlockSpec((_TN, F), lambda r, n: (n, 0)),
            pl.BlockSpec((1, F, H), lambda r, n: (r, 0, 0)),
        ],
        out_specs=pl.BlockSpec((1, _TN, H), lambda r, n: (r, n, 0)),
        out_shape=jax.ShapeDtypeStruct((R, N, H), jnp.float32),
    )(x, Wr)


def _root(x, Wroot, b):
    N = x.shape[0]
    H = Wroot.shape[1]
    return pl.pallas_call(
        _root_krn,
        out_shape=jax.ShapeDtypeStruct((N, H), jnp.float32),
    )(x, Wroot, b.reshape(1, H))


def _count(N, nch, dst_c, et_c, w0_c):
    smem = pl.BlockSpec((1, 1, _CS), lambda c: (c, 0, 0),
                        memory_space=pltpu.SMEM)
    return pl.pallas_call(
        functools.partial(_count_krn, nch),
        grid=(nch,),
        in_specs=[smem, smem, smem],
        out_specs=pl.BlockSpec((N, _LANES), lambda c: (0, 0)),
        out_shape=jax.ShapeDtypeStruct((N, _LANES), jnp.float32),
    )(dst_c, et_c, w0_c)


def _scatter(nch, rel_map, h_all, out0, cnt, src_c, dst_c, et_c, w0_c):
    R, N, H = h_all.shape
    smem = pl.BlockSpec((1, 1, _CS), lambda c, rm: (c, 0, 0),
                        memory_space=pltpu.SMEM)
    grid_spec = pltpu.PrefetchScalarGridSpec(
        num_scalar_prefetch=1,
        grid=(nch,),
        in_specs=[
            pl.BlockSpec((1, N, H), lambda c, rm: (rm[c], 0, 0)),
            pl.BlockSpec((N, H), lambda c, rm: (0, 0)),
            pl.BlockSpec((N, _LANES), lambda c, rm: (0, 0)),
            smem, smem, smem, smem,
        ],
        out_specs=pl.BlockSpec((N, H), lambda c, rm: (0, 0)),
    )
    return pl.pallas_call(
        functools.partial(_scatter_krn, nch),
        grid_spec=grid_spec,
        out_shape=jax.ShapeDtypeStruct((N, H), jnp.float32),
    )(rel_map, h_all, out0, cnt, src_c, dst_c, et_c, w0_c)


def _pool(h, batch2d):
    N, H = h.shape
    return pl.pallas_call(
        _pool_krn,
        grid=(_G,),
        in_specs=[
            pl.BlockSpec((N, H), lambda g: (0, 0)),
            pl.BlockSpec((N, _LANES), lambda g: (0, 0)),
        ],
        out_specs=pl.BlockSpec((_G, H), lambda g: (0, 0)),
        out_shape=jax.ShapeDtypeStruct((_G, H), jnp.float32),
    )(h, batch2d)


def kernel(x, edge_index, edge_type, batch, Wr1, Wroot1, b1, Wr2, Wroot2, b2):
    N, F = x.shape
    E = edge_type.shape[0]
    R = Wr1.shape[0]
    nch = -(-E // _CS) + R
    EP = nch * _CS

    order = jnp.argsort(edge_type)
    et_s = edge_type[order]
    src_s = edge_index[0][order]
    dst_s = edge_index[1][order]

    rel_ids = jnp.arange(R, dtype=jnp.int32)
    cnt_r = jnp.sum(et_s[None, :] == rel_ids[:, None], axis=1).astype(jnp.int32)
    chunks_r = (cnt_r + _CS - 1) // _CS
    pad_chunk_start = jnp.concatenate(
        [jnp.zeros((1,), jnp.int32), jnp.cumsum(chunks_r)]
    )
    pad_start = pad_chunk_start[:-1] * _CS
    first_sorted = jnp.concatenate(
        [jnp.zeros((1,), jnp.int32), jnp.cumsum(cnt_r)]
    )[:-1]
    pos = pad_start[et_s] + (jnp.arange(E, dtype=jnp.int32) - first_sorted[et_s])

    src_c = jnp.zeros((EP,), jnp.int32).at[pos].set(src_s).reshape(nch, 1, _CS)
    dst_c = jnp.zeros((EP,), jnp.int32).at[pos].set(dst_s).reshape(nch, 1, _CS)
    et_c = jnp.zeros((EP,), jnp.int32).at[pos].set(et_s).reshape(nch, 1, _CS)
    w0_c = (
        jnp.zeros((EP,), jnp.float32).at[pos].set(1.0).reshape(nch, 1, _CS)
    )
    rel_map = jnp.clip(
        jnp.searchsorted(pad_chunk_start[1:], jnp.arange(nch, dtype=jnp.int32),
                         side="right"),
        0, R - 1,
    ).astype(jnp.int32)

    cnt = _count(N, nch, dst_c, et_c, w0_c)

    h_all1 = _transform(x, Wr1)
    out01 = _root(x, Wroot1, b1)
    h1 = _scatter(nch, rel_map, h_all1, out01, cnt, src_c, dst_c, et_c, w0_c)

    h_all2 = _transform(h1, Wr2)
    out02 = _root(h1, Wroot2, b2)
    h2 = _scatter(nch, rel_map, h_all2, out02, cnt, src_c, dst_c, et_c, w0_c)

    batch2d = jnp.broadcast_to(
        batch.astype(jnp.int32)[:, None], (N, _LANES)
    )
    pooled = _pool(h2, batch2d)
    return (h2, pooled)
